# baseline (speedup 1.0000x reference)
import functools

import jax
import jax.numpy as jnp
from jax.experimental import pallas as pl

_N = 8192
_D_IN = 2048
_D_OUT = 2048
_H1 = 100
_H2 = 50
_EPS = 1e-5
_BLOCK = 512


def _mlp_block(x_ref, w1_ref, b1_ref, w2_ref, b2_ref, w3_ref, b3_ref,
               m0_ref, v0_ref, m1_ref, v1_ref, m2_ref, v2_ref, out_ref):
    xn = (x_ref[...] - m0_ref[...]) * jax.lax.rsqrt(v0_ref[...] + _EPS)
    h = jax.lax.dot_general(xn, w1_ref[...], (((1,), (1,)), ((), ())),
                            preferred_element_type=jnp.float32)
    h = h + b1_ref[...]
    h = jnp.maximum(h, 0.0)
    h = (h - m1_ref[...]) * jax.lax.rsqrt(v1_ref[...] + _EPS)
    g = jax.lax.dot_general(h, w2_ref[...], (((1,), (1,)), ((), ())),
                            preferred_element_type=jnp.float32)
    g = g + b2_ref[...]
    g = jnp.maximum(g, 0.0)
    g = (g - m2_ref[...]) * jax.lax.rsqrt(v2_ref[...] + _EPS)
    out = jax.lax.dot_general(g, w3_ref[...], (((1,), (1,)), ((), ())),
                              preferred_element_type=jnp.float32)
    out_ref[...] = out + b3_ref[...]


@functools.partial(jax.jit, static_argnames=("interpret",))
def kernel(x, W1, b1, W2, b2, W3, b3, bn0_mean, bn0_var, bn1_mean, bn1_var,
           bn2_mean, bn2_var, interpret=False):
    n = x.shape[0]
    grid = (n // _BLOCK,)

    def row_blk(i):
        return (i, 0)

    def const_blk(i):
        return (0, 0)

    full = lambda shape: pl.BlockSpec(shape, const_blk)

    return pl.pallas_call(
        _mlp_block,
        grid=grid,
        in_specs=[
            pl.BlockSpec((_BLOCK, _D_IN), row_blk),
            full((_H1, _D_IN)),
            full((1, _H1)),
            full((_H2, _H1)),
            full((1, _H2)),
            full((_D_OUT, _H2)),
            full((1, _D_OUT)),
            full((1, _D_IN)),
            full((1, _D_IN)),
            full((1, _H1)),
            full((1, _H1)),
            full((1, _H2)),
            full((1, _H2)),
        ],
        out_specs=pl.BlockSpec((_BLOCK, _D_OUT), row_blk),
        out_shape=jax.ShapeDtypeStruct((n, _D_OUT), jnp.float32),
        interpret=interpret,
    )(x, W1, b1.reshape(1, -1), W2, b2.reshape(1, -1), W3,
      b3.reshape(1, -1), bn0_mean.reshape(1, -1), bn0_var.reshape(1, -1),
      bn1_mean.reshape(1, -1), bn1_var.reshape(1, -1),
      bn2_mean.reshape(1, -1), bn2_var.reshape(1, -1))
